# baseline (speedup 1.0000x reference)
import functools

import jax
import jax.numpy as jnp
from jax import lax
from jax.experimental import pallas as pl
from jax.experimental.pallas import tpu as pltpu
from jax.experimental.pallas import tpu_sc as plsc

N_NODES = 10000
N_EDGES = 320000
D_FEAT = 128
N_HID = 16
N_CLASS = 40
F2 = 48

NW = 32
EPAD = 327680
EPW = EPAD // NW
NPAD = 10240
RPT = NPAD // 16


def _make_spmm(F, K):
  mesh = plsc.VectorSubcoreMesh(core_axis_name="c", subcore_axis_name="s")

  NBUF = 3
  CPW = EPW // K
  ROWS_PER_CHUNK = K // 128

  @functools.partial(
      pl.kernel,
      out_type=jax.ShapeDtypeStruct((2, NPAD, F), jnp.float32),
      mesh=mesh,
      compiler_params=pltpu.CompilerParams(
          needs_layout_passes=False, use_tc_tiling_on_sc=False,
          disable_bounds_checks=True),
      scratch_types=[
          [pltpu.VMEM((ROWS_PER_CHUNK, 128), jnp.int32)] * NBUF,
          [pltpu.VMEM((ROWS_PER_CHUNK, 128), jnp.int32)] * NBUF,
          [pltpu.VMEM((K,), jnp.float32)] * NBUF,
          [pltpu.VMEM((K, F), jnp.float32)] * NBUF,
          pltpu.VMEM_SHARED((NPAD, F), jnp.float32),
          [pltpu.SemaphoreType.DMA] * NBUF,
          [pltpu.SemaphoreType.DMA] * NBUF,
      ],
  )
  def spmm(h_hbm, src_hbm, dst_hbm, vals_hbm, out_hbm,
           src_v, dst_v, vals_v, rows_v, acc_sh, gsem, ssem):
    cid = lax.axis_index("c")
    sid = lax.axis_index("s")
    wid = cid * 16 + sid

    zero = jnp.zeros((16,), jnp.float32)
    legs = []
    off = 0
    while off < RPT:
      legs.append((off, min(K, RPT - off)))
      off += min(K, RPT - off)

    def zero_body(r, carry):
      for f in range(F // 16):
        rows_v[0][r, pl.ds(f * 16, 16)] = zero
      return carry

    lax.fori_loop(0, min(K, RPT), zero_body, None)
    for off, n in legs:
      pltpu.sync_copy(rows_v[0].at[pl.ds(0, n)],
                      acc_sh.at[pl.ds(sid * RPT + off, n)])
    plsc.subcore_barrier()

    def load_and_gather(c, b):
      rb = wid * (EPW // 128) + c * ROWS_PER_CHUNK
      base = wid * EPW + c * K
      pltpu.sync_copy(src_hbm.at[pl.ds(rb, ROWS_PER_CHUNK)], src_v[b])
      pltpu.sync_copy(dst_hbm.at[pl.ds(rb, ROWS_PER_CHUNK)], dst_v[b])
      pltpu.sync_copy(vals_hbm.at[pl.ds(base, K)], vals_v[b])
      for r in range(ROWS_PER_CHUNK):
        pltpu.async_copy(h_hbm.at[src_v[b].at[r]],
                         rows_v[b].at[pl.ds(r * 128, 128)], gsem[b])

    def wait_gather(b):
      for r in range(ROWS_PER_CHUNK):
        pltpu.make_async_copy(h_hbm.at[src_v[b].at[r]],
                              rows_v[b].at[pl.ds(r * 128, 128)],
                              gsem[b]).wait()

    def scale_rows(b):
      idx_consts = [jnp.full((16,), j, jnp.int32) for j in range(16)]

      def mul_body(g, carry):
        v16 = vals_v[b][pl.ds(g * 16, 16)]
        for j in range(16):
          e = g * 16 + j
          splat = jnp.take_along_axis(v16, idx_consts[j], axis=0,
                                      mode="promise_in_bounds")
          for f in range(F // 16):
            sl = pl.ds(f * 16, 16)
            rows_v[b][e, sl] = rows_v[b][e, sl] * splat
        return carry

      lax.fori_loop(0, K // 16, mul_body, None)

    def start_scatter(b):
      for r in range(ROWS_PER_CHUNK):
        pltpu.async_copy(rows_v[b].at[pl.ds(r * 128, 128)],
                         acc_sh.at[dst_v[b].at[r]], ssem[b], add=True)

    def wait_scatter(b):
      for r in range(ROWS_PER_CHUNK):
        pltpu.make_async_copy(rows_v[b].at[pl.ds(r * 128, 128)],
                              acc_sh.at[dst_v[b].at[r]], ssem[b]).wait()

    load_and_gather(0, 0)
    load_and_gather(1, 1)
    for c in range(CPW):
      p = c % NBUF
      wait_gather(p)
      scale_rows(p)
      start_scatter(p)
      if c + 2 < CPW:
        b = (c + 2) % NBUF
        if c >= 1:
          wait_scatter(b)
        load_and_gather(c + 2, b)
    for c in (CPW - 3, CPW - 2, CPW - 1):
      wait_scatter(c % NBUF)
    plsc.subcore_barrier()

    for off, n in legs:
      pltpu.sync_copy(acc_sh.at[pl.ds(sid * RPT + off, n)],
                      rows_v[0].at[pl.ds(0, n)])
      pltpu.sync_copy(rows_v[0].at[pl.ds(0, n)],
                      out_hbm.at[cid, pl.ds(sid * RPT + off, n)])

  return spmm


F1 = 32
_spmm32 = _make_spmm(F1, 512)
_spmm16 = _make_spmm(N_HID, 1024)


RB = 2000
NRB = N_NODES // RB


def _linear1(x, W1p, b1p):
  def body(x_ref, w_ref, b_ref, o_ref):
    o_ref[...] = jnp.dot(x_ref[...], w_ref[...],
                         preferred_element_type=jnp.float32) + b_ref[...]

  return pl.pallas_call(
      body,
      grid=(NRB,),
      in_specs=[
          pl.BlockSpec((RB, D_FEAT), lambda i: (i, 0)),
          pl.BlockSpec((D_FEAT, F1), lambda i: (0, 0)),
          pl.BlockSpec((1, F1), lambda i: (0, 0)),
      ],
      out_specs=pl.BlockSpec((RB, F1), lambda i: (i, 0)),
      out_shape=jax.ShapeDtypeStruct((N_NODES, F1), jnp.float32),
  )(x, W1p, b1p)


def _relu_sum(p):
  def body(p_ref, o_ref):
    o_ref[...] = jnp.maximum(p_ref[0, :, :N_HID] + p_ref[1, :, :N_HID], 0.0)

  return pl.pallas_call(
      body,
      grid=(NRB,),
      in_specs=[pl.BlockSpec((2, RB, F1), lambda i: (0, i, 0))],
      out_specs=pl.BlockSpec((RB, N_HID), lambda i: (i, 0)),
      out_shape=jax.ShapeDtypeStruct((N_NODES, N_HID), jnp.float32),
  )(p)


def _linear2_log_softmax(p1, p2, W2p, b2p):
  def body(p1_ref, p2_ref, w_ref, b_ref, o_ref):
    q = p2_ref[0] + p2_ref[1]
    deg = (p1_ref[0, :, N_HID:N_HID + 1] + p1_ref[1, :, N_HID:N_HID + 1])
    z = (jnp.dot(q, w_ref[...], preferred_element_type=jnp.float32)
         + deg * b_ref[...])
    col = lax.broadcasted_iota(jnp.int32, (RB, F2), 1)
    zm = jnp.where(col < N_CLASS, z, -jnp.inf)
    m = jnp.max(zm, axis=1, keepdims=True)
    ez = jnp.exp(zm - m)
    lse = jnp.log(jnp.sum(ez, axis=1, keepdims=True)) + m
    o_ref[...] = (z - lse)[:, :N_CLASS]

  return pl.pallas_call(
      body,
      grid=(NRB,),
      in_specs=[
          pl.BlockSpec((2, RB, F1), lambda i: (0, i, 0)),
          pl.BlockSpec((2, RB, N_HID), lambda i: (0, i, 0)),
          pl.BlockSpec((N_HID, F2), lambda i: (0, 0)),
          pl.BlockSpec((1, F2), lambda i: (0, 0)),
      ],
      out_specs=pl.BlockSpec((RB, N_CLASS), lambda i: (i, 0)),
      out_shape=jax.ShapeDtypeStruct((N_NODES, N_CLASS), jnp.float32),
  )(p1, p2, W2p, b2p)


def kernel(x, edge_index, adj_vals, W1, b1, W2, b2):
  ei = edge_index.astype(jnp.int32)
  pad = EPAD - N_EDGES
  pad_ar = jnp.arange(pad, dtype=jnp.int32)
  dst = jnp.concatenate([ei[0], N_NODES + pad_ar % (NPAD - N_NODES)])
  src = jnp.concatenate([ei[1], pad_ar % N_NODES])
  vals = jnp.concatenate([adj_vals.astype(jnp.float32),
                          jnp.zeros((pad,), jnp.float32)])
  src2 = src.reshape(EPAD // 128, 128)
  dst2 = dst.reshape(EPAD // 128, 128)

  W1p = jnp.zeros((D_FEAT, F1), jnp.float32).at[:, :N_HID].set(W1)
  b1p = (jnp.zeros((1, F1), jnp.float32).at[0, :N_HID].set(b1)
         .at[0, N_HID].set(1.0))
  W2p = jnp.zeros((N_HID, F2), jnp.float32).at[:, :N_CLASS].set(W2)
  b2p = jnp.zeros((1, F2), jnp.float32).at[0, :N_CLASS].set(b2)

  h32 = _linear1(x, W1p, b1p)
  p1 = _spmm32(h32, src2, dst2, vals)
  relu_h = _relu_sum(p1)
  p2 = _spmm16(relu_h, src2, dst2, vals)
  return _linear2_log_softmax(p1, p2, W2p, b2p)


# speedup vs baseline: 1.1899x; 1.1899x over previous
import functools

import jax
import jax.numpy as jnp
from jax import lax
from jax.experimental import pallas as pl
from jax.experimental.pallas import tpu as pltpu
from jax.experimental.pallas import tpu_sc as plsc

N_NODES = 10000
N_EDGES = 320000
D_FEAT = 128
N_HID = 16
N_CLASS = 40
F2 = 48

NW = 32
EPAD = 327680
EPW = EPAD // NW
NPAD = 10240
RPT = NPAD // 16


def _make_spmm(F, K):
  mesh = plsc.VectorSubcoreMesh(core_axis_name="c", subcore_axis_name="s")

  NBUF = 3
  CPW = EPW // K
  ROWS_PER_CHUNK = K // 128
  SLAB = EPW // 128

  @functools.partial(
      pl.kernel,
      out_type=jax.ShapeDtypeStruct((2, NPAD, F), jnp.float32),
      mesh=mesh,
      compiler_params=pltpu.CompilerParams(
          needs_layout_passes=False, use_tc_tiling_on_sc=False,
          disable_bounds_checks=True),
      scratch_types=[
          pltpu.VMEM((3, SLAB, 128), jnp.int32),
          [pltpu.VMEM((K, F), jnp.float32)] * NBUF,
          pltpu.VMEM_SHARED((NPAD, F), jnp.float32),
          [pltpu.SemaphoreType.DMA] * NBUF,
          [pltpu.SemaphoreType.DMA] * NBUF,
      ],
  )
  def spmm(h_hbm, edges_hbm, out_hbm, edges_v, rows_v, acc_sh, gsem, ssem):
    cid = lax.axis_index("c")
    sid = lax.axis_index("s")
    wid = cid * 16 + sid

    for plane in range(3):
      pltpu.sync_copy(edges_hbm.at[plane, pl.ds(wid * SLAB, SLAB)],
                      edges_v.at[plane])

    zero = jnp.zeros((16,), jnp.float32)
    legs = []
    off = 0
    while off < RPT:
      legs.append((off, min(K, RPT - off)))
      off += min(K, RPT - off)

    def zero_body(r, carry):
      for f in range(F // 16):
        rows_v[0][r, pl.ds(f * 16, 16)] = zero
      return carry

    lax.fori_loop(0, min(K, RPT), zero_body, None)
    for off, n in legs:
      pltpu.sync_copy(rows_v[0].at[pl.ds(0, n)],
                      acc_sh.at[pl.ds(sid * RPT + off, n)])
    plsc.subcore_barrier()

    def start_gather(c, b):
      rb = c * ROWS_PER_CHUNK
      for r in range(ROWS_PER_CHUNK):
        pltpu.async_copy(h_hbm.at[edges_v.at[0, rb + r]],
                         rows_v[b].at[pl.ds(r * 128, 128)], gsem[b])

    def wait_gather(c, b):
      rb = c * ROWS_PER_CHUNK
      for r in range(ROWS_PER_CHUNK):
        pltpu.make_async_copy(h_hbm.at[edges_v.at[0, rb + r]],
                              rows_v[b].at[pl.ds(r * 128, 128)],
                              gsem[b]).wait()

    def scale_rows(c, b):
      idx_consts = [jnp.full((16,), j, jnp.int32) for j in range(16)]

      def mul_body(r2, carry):
        for o8 in range(8):
          v16 = plsc.bitcast(
              edges_v[2, c * ROWS_PER_CHUNK + r2, pl.ds(o8 * 16, 16)],
              jnp.float32)
          for j in range(16):
            e = r2 * 128 + o8 * 16 + j
            splat = jnp.take_along_axis(v16, idx_consts[j], axis=0,
                                        mode="promise_in_bounds")
            for f in range(F // 16):
              sl = pl.ds(f * 16, 16)
              rows_v[b][e, sl] = rows_v[b][e, sl] * splat
        return carry

      lax.fori_loop(0, ROWS_PER_CHUNK, mul_body, None)

    def start_scatter(c, b):
      rb = c * ROWS_PER_CHUNK
      for r in range(ROWS_PER_CHUNK):
        pltpu.async_copy(rows_v[b].at[pl.ds(r * 128, 128)],
                         acc_sh.at[edges_v.at[1, rb + r]], ssem[b], add=True)

    def wait_scatter(c, b):
      rb = c * ROWS_PER_CHUNK
      for r in range(ROWS_PER_CHUNK):
        pltpu.make_async_copy(rows_v[b].at[pl.ds(r * 128, 128)],
                              acc_sh.at[edges_v.at[1, rb + r]],
                              ssem[b]).wait()

    start_gather(0, 0)
    start_gather(1, 1)
    for c in range(CPW):
      p = c % NBUF
      wait_gather(c, p)
      scale_rows(c, p)
      start_scatter(c, p)
      if c + 2 < CPW:
        b = (c + 2) % NBUF
        if c >= 1:
          wait_scatter(c - 1, b)
        start_gather(c + 2, b)
    for c in (CPW - 3, CPW - 2, CPW - 1):
      wait_scatter(c, c % NBUF)
    plsc.subcore_barrier()

    for off, n in legs:
      pltpu.sync_copy(acc_sh.at[pl.ds(sid * RPT + off, n)],
                      rows_v[0].at[pl.ds(0, n)])
      pltpu.sync_copy(rows_v[0].at[pl.ds(0, n)],
                      out_hbm.at[cid, pl.ds(sid * RPT + off, n)])

  return spmm


F1 = 32
_spmm32 = _make_spmm(F1, 512)
_spmm16 = _make_spmm(N_HID, 1024)


RB = 2000
NRB = N_NODES // RB


def _linear1(x, W1p, b1p):
  def body(x_ref, w_ref, b_ref, o_ref):
    o_ref[...] = jnp.dot(x_ref[...], w_ref[...],
                         preferred_element_type=jnp.float32) + b_ref[...]

  return pl.pallas_call(
      body,
      grid=(NRB,),
      in_specs=[
          pl.BlockSpec((RB, D_FEAT), lambda i: (i, 0)),
          pl.BlockSpec((D_FEAT, F1), lambda i: (0, 0)),
          pl.BlockSpec((1, F1), lambda i: (0, 0)),
      ],
      out_specs=pl.BlockSpec((RB, F1), lambda i: (i, 0)),
      out_shape=jax.ShapeDtypeStruct((N_NODES, F1), jnp.float32),
  )(x, W1p, b1p)


def _relu_sum(p):
  def body(p_ref, o_ref):
    o_ref[...] = jnp.maximum(p_ref[0, :, :N_HID] + p_ref[1, :, :N_HID], 0.0)

  return pl.pallas_call(
      body,
      grid=(NRB,),
      in_specs=[pl.BlockSpec((2, RB, F1), lambda i: (0, i, 0))],
      out_specs=pl.BlockSpec((RB, N_HID), lambda i: (i, 0)),
      out_shape=jax.ShapeDtypeStruct((N_NODES, N_HID), jnp.float32),
  )(p)


def _linear2_log_softmax(p1, p2, W2p, b2p):
  def body(p1_ref, p2_ref, w_ref, b_ref, o_ref):
    q = p2_ref[0] + p2_ref[1]
    deg = (p1_ref[0, :, N_HID:N_HID + 1] + p1_ref[1, :, N_HID:N_HID + 1])
    z = (jnp.dot(q, w_ref[...], preferred_element_type=jnp.float32)
         + deg * b_ref[...])
    col = lax.broadcasted_iota(jnp.int32, (RB, F2), 1)
    zm = jnp.where(col < N_CLASS, z, -jnp.inf)
    m = jnp.max(zm, axis=1, keepdims=True)
    ez = jnp.exp(zm - m)
    lse = jnp.log(jnp.sum(ez, axis=1, keepdims=True)) + m
    o_ref[...] = (z - lse)[:, :N_CLASS]

  return pl.pallas_call(
      body,
      grid=(NRB,),
      in_specs=[
          pl.BlockSpec((2, RB, F1), lambda i: (0, i, 0)),
          pl.BlockSpec((2, RB, N_HID), lambda i: (0, i, 0)),
          pl.BlockSpec((N_HID, F2), lambda i: (0, 0)),
          pl.BlockSpec((1, F2), lambda i: (0, 0)),
      ],
      out_specs=pl.BlockSpec((RB, N_CLASS), lambda i: (i, 0)),
      out_shape=jax.ShapeDtypeStruct((N_NODES, N_CLASS), jnp.float32),
  )(p1, p2, W2p, b2p)


def _pack_edges(ei, av):
  pad = EPAD - N_EDGES

  def body(e_ref, v_ref, o_ref):
    ar = lax.broadcasted_iota(jnp.int32, (pad,), 0)
    src = jnp.concatenate([e_ref[1], ar % N_NODES])
    dst = jnp.concatenate([e_ref[0], N_NODES + ar % (NPAD - N_NODES)])
    vi = lax.bitcast_convert_type(v_ref[...], jnp.int32)
    vip = jnp.concatenate([vi, jnp.zeros((pad,), jnp.int32)])
    o_ref[0] = src.reshape(EPAD // 128, 128)
    o_ref[1] = dst.reshape(EPAD // 128, 128)
    o_ref[2] = vip.reshape(EPAD // 128, 128)

  return pl.pallas_call(
      body,
      out_shape=jax.ShapeDtypeStruct((3, EPAD // 128, 128), jnp.int32),
  )(ei, av)


def kernel(x, edge_index, adj_vals, W1, b1, W2, b2):
  ei = edge_index.astype(jnp.int32)
  av = adj_vals.astype(jnp.float32)
  edges = _pack_edges(ei, av)

  W1p = jnp.zeros((D_FEAT, F1), jnp.float32).at[:, :N_HID].set(W1)
  b1p = (jnp.zeros((1, F1), jnp.float32).at[0, :N_HID].set(b1)
         .at[0, N_HID].set(1.0))
  W2p = jnp.zeros((N_HID, F2), jnp.float32).at[:, :N_CLASS].set(W2)
  b2p = jnp.zeros((1, F2), jnp.float32).at[0, :N_CLASS].set(b2)

  h32 = _linear1(x, W1p, b1p)
  p1 = _spmm32(h32, edges)
  relu_h = _relu_sum(p1)
  p2 = _spmm16(relu_h, edges)
  return _linear2_log_softmax(p1, p2, W2p, b2p)
